# baseline (speedup 1.0000x reference)
import jax
import jax.numpy as jnp
from jax.experimental import pallas as pl
from jax.experimental.pallas import tpu as pltpu

B = 1024
NUM_FEATURES = 128
NUM_CLASSES = 100000
MBLK = 2048


def _mm_kernel(m_ref, x_ref, o_ref):
    o_ref[...] = jax.lax.dot_general(
        m_ref[...].astype(jnp.bfloat16), x_ref[...],
        dimension_numbers=(((1,), (1,)), ((), ())),
        preferred_element_type=jnp.float32,
    )


def kernel(inputs, targets, epoch, mem):
    del targets, epoch
    logits_t = pl.pallas_call(
        _mm_kernel,
        grid=(pl.cdiv(NUM_CLASSES, MBLK),),
        in_specs=[
            pl.BlockSpec((MBLK, NUM_FEATURES), lambda j: (j, 0)),
            pl.BlockSpec((B, NUM_FEATURES), lambda j: (0, 0)),
        ],
        out_specs=pl.BlockSpec((MBLK, B), lambda j: (j, 0)),
        out_shape=jax.ShapeDtypeStruct((NUM_CLASSES, B), jnp.float32),
        compiler_params=pltpu.CompilerParams(
            dimension_semantics=("parallel",),
        ),
    )(mem, inputs.astype(jnp.bfloat16))
    return logits_t.T


# speedup vs baseline: 1.0200x; 1.0200x over previous
import jax
import jax.numpy as jnp
from jax.experimental import pallas as pl
from jax.experimental.pallas import tpu as pltpu

B = 1024
NUM_FEATURES = 128
NUM_CLASSES = 100000
MBLK = 4096


def _mm_kernel(m_ref, x_ref, o_ref):
    o_ref[...] = jax.lax.dot_general(
        m_ref[...].astype(jnp.bfloat16), x_ref[...],
        dimension_numbers=(((1,), (1,)), ((), ())),
        preferred_element_type=jnp.float32,
    )


def kernel(inputs, targets, epoch, mem):
    del targets, epoch
    logits_t = pl.pallas_call(
        _mm_kernel,
        grid=(pl.cdiv(NUM_CLASSES, MBLK),),
        in_specs=[
            pl.BlockSpec((MBLK, NUM_FEATURES), lambda j: (j, 0)),
            pl.BlockSpec((B, NUM_FEATURES), lambda j: (0, 0)),
        ],
        out_specs=pl.BlockSpec((MBLK, B), lambda j: (j, 0)),
        out_shape=jax.ShapeDtypeStruct((NUM_CLASSES, B), jnp.float32),
        compiler_params=pltpu.CompilerParams(
            dimension_semantics=("parallel",),
        ),
    )(mem, inputs.astype(jnp.bfloat16))
    return logits_t.T


# speedup vs baseline: 1.0248x; 1.0047x over previous
import jax
import jax.numpy as jnp
from jax.experimental import pallas as pl
from jax.experimental.pallas import tpu as pltpu

B = 1024
NUM_FEATURES = 128
NUM_CLASSES = 100000
MBLK = 6144


def _mm_kernel(m_ref, x_ref, o_ref):
    o_ref[...] = jax.lax.dot_general(
        m_ref[...].astype(jnp.bfloat16), x_ref[...],
        dimension_numbers=(((1,), (1,)), ((), ())),
        preferred_element_type=jnp.float32,
    )


def kernel(inputs, targets, epoch, mem):
    del targets, epoch
    logits_t = pl.pallas_call(
        _mm_kernel,
        grid=(pl.cdiv(NUM_CLASSES, MBLK),),
        in_specs=[
            pl.BlockSpec((MBLK, NUM_FEATURES), lambda j: (j, 0)),
            pl.BlockSpec((B, NUM_FEATURES), lambda j: (0, 0)),
        ],
        out_specs=pl.BlockSpec((MBLK, B), lambda j: (j, 0)),
        out_shape=jax.ShapeDtypeStruct((NUM_CLASSES, B), jnp.float32),
        compiler_params=pltpu.CompilerParams(
            dimension_semantics=("parallel",),
        ),
    )(mem, inputs.astype(jnp.bfloat16))
    return logits_t.T
